# baseline (speedup 1.0000x reference)
import jax
import jax.numpy as jnp
from math import pi
from jax.experimental import pallas as pl
from jax.experimental.pallas import tpu as pltpu

V_SIZE = 120
DV = 10.0
VLIM = 500.0
F32 = jnp.float32


def _dot(a, b, dn):
    return jax.lax.dot_general(a, b, dn, preferred_element_type=F32)


_DN_TA = (((0,), (0,)), ((), ()))
_DN_NN = (((1,), (0,)), ((), ()))
_DN_TB = (((1,), (1,)), ((), ()))


def _pool_rows(t):
    t3 = t.reshape(t.shape[0] // 2, 2, t.shape[1])
    return jnp.maximum(t3[:, 0, :], t3[:, 1, :])


def _maxpool(t, side):
    tx = _pool_rows(t)
    h = side // 2
    t4 = tx.reshape(side // 2, 2, h, t.shape[1])
    ty = jnp.maximum(t4[:, 0, :, :], t4[:, 1, :, :])
    return ty.reshape(h * h, t.shape[1])


def _conv_combine(scr, pad, side, nrows, cout, mask_notlast, mask_notfirst):
    acc = None
    for dy in (-1, 0, 1):
        for dx in (-1, 0, 1):
            k = (dy + 1) * 3 + (dx + 1)
            start = pad + dy * side + dx
            sl = scr[pl.ds(start, nrows), k * cout:(k + 1) * cout]
            if dx == 1:
                sl = jnp.where(mask_notlast, sl, 0.0)
            elif dx == -1:
                sl = jnp.where(mask_notfirst, sl, 0.0)
            acc = sl if acc is None else acc + sl
    return acc


def _xmask(side, nrows, cout):
    xs = jax.lax.broadcasted_iota(jnp.int32, (nrows, cout), 0) & (side - 1)
    return xs != (side - 1), xs != 0


def _conv_block(h, wh, bias, scr, pad, side, cout, dn, first_step):
    nrows = side * side
    part = _dot(h, wh, dn)

    @pl.when(first_step)
    def _():
        scr[0:pad, :] = jnp.zeros((pad, 9 * cout), F32)
        scr[pad + nrows:pad + nrows + pad, :] = jnp.zeros((pad, 9 * cout), F32)

    scr[pl.ds(pad, nrows), :] = part
    mnl, mnf = _xmask(side, nrows, cout)
    acc = _conv_combine(scr, pad, side, nrows, cout, mnl, mnf)
    r = jax.nn.relu(acc + bias)
    return _maxpool(r, side)


def _enc_kernel(x_ref, w1_ref, w2_ref, w3_ref,
                b1_ref, b2_ref, b3_ref, out_ref, scr1, scr2, scr3):
    first = pl.program_id(0) == 0
    xc = x_ref[0]
    h1 = _conv_block(xc, w1_ref[...], b1_ref[...],
                     scr1, 72, 64, 32, _DN_TA, first)
    h2 = _conv_block(h1, w2_ref[...], b2_ref[...],
                     scr2, 40, 32, 64, _DN_NN, first)
    h3 = _conv_block(h2, w3_ref[...], b3_ref[...],
                     scr3, 24, 16, 128, _DN_NN, first)
    out_ref[0] = h3.T


def _fc_kernel(h_ref, w1_ref, b1_ref, w2_ref, b2_ref, w3_ref, b3_ref,
               w4_ref, b4_ref, w5_ref, b5_ref, lat_ref, y1_scr):
    j = pl.program_id(0)
    y = _dot(h_ref[...], w1_ref[...], _DN_TB) + b1_ref[...]
    y1_scr[:, pl.ds(j * 256, 256)] = jax.nn.relu(y)

    @pl.when(j == 7)
    def _():
        y1 = y1_scr[...]
        y2 = jax.nn.relu(_dot(y1, w2_ref[...], _DN_TB) + b2_ref[...])
        y3 = jax.nn.relu(_dot(y2, w3_ref[...], _DN_TB) + b3_ref[...])
        y4 = jax.nn.relu(_dot(y3, w4_ref[...], _DN_TB) + b4_ref[...])
        y5 = _dot(y4, w5_ref[...], _DN_TB) + b5_ref[...]
        lat_ref[...] = jnp.clip(y5, -1.0, 1.0)


def _atan(x):
    ax = jnp.abs(x)
    big = ax > 2.414213562373095
    mid = ax > 0.41421356237309503
    arg = jnp.where(big, -1.0 / ax, jnp.where(mid, (ax - 1.0) / (ax + 1.0), ax))
    u = arg * arg
    y = ((((8.05374449538e-2 * u - 1.38776856032e-1) * u + 1.99777106478e-1) * u
          - 3.33329491539e-1) * u * arg + arg)
    y = y + jnp.where(big, jnp.float32(pi / 2), jnp.where(mid, jnp.float32(pi / 4), 0.0))
    return jnp.where(x < 0, -y, y)


def _cube_kernel(d_ref, xx_ref, yy_ref, cube_ref, v_ref):
    cp = d_ref[0, 0, 0]
    sp = d_ref[0, 0, 1]
    ci = d_ref[0, 0, 2]
    si = d_ref[0, 0, 3]
    a = d_ref[0, 0, 4]
    ah = d_ref[0, 0, 5]
    vh = d_ref[0, 0, 6]
    xx = xx_ref[...]
    yy = yy_ref[...]
    xx_t = xx * cp + yy * sp
    yy_t = -xx * sp + yy * (ci * cp)
    rr = jnp.sqrt(xx_t * xx_t + yy_t * yy_t)
    sb = jnp.exp(-rr / a)
    vel = jnp.sqrt(vh * vh * (1.0 - ah / rr * _atan(rr / ah)))
    vel = vel * VLIM
    vel = vel * (-(xx_t * cp + yy_t * sp) / rr * si)
    v_ref[0] = vel
    b = jnp.clip(jnp.floor(vel / DV) + V_SIZE // 2, 0.0, float(V_SIZE))
    bi = b.astype(jnp.int32)
    bins = jax.lax.broadcasted_iota(jnp.int32, (V_SIZE + 1, 32, 128), 0)
    cube_ref[0] = jnp.where(bi[None, :, :] == bins, sb[None, :, :], 0.0)


def kernel(x, xx, yy, c1w, c1b, c2w, c2b, c3w, c3b,
           l1w, l1b, l2w, l2b, l3w, l3b, l4w, l4b, l5w, l5b):
    B = x.shape[0]
    x4 = x.reshape(B, 120, 4096)

    def expand(w):
        co, ci = w.shape[0], w.shape[1]
        return jnp.transpose(w, (1, 2, 3, 0)).reshape(ci, 9 * co)

    w1e = expand(c1w)
    w2e = expand(c2w)
    w3e = expand(c3w)

    h3 = pl.pallas_call(
        _enc_kernel,
        out_shape=jax.ShapeDtypeStruct((B, 128, 64), F32),
        grid=(B,),
        in_specs=[
            pl.BlockSpec((1, 120, 4096), lambda i: (i, 0, 0)),
            pl.BlockSpec((120, 288), lambda i: (0, 0)),
            pl.BlockSpec((32, 576), lambda i: (0, 0)),
            pl.BlockSpec((64, 1152), lambda i: (0, 0)),
            pl.BlockSpec((1, 32), lambda i: (0, 0)),
            pl.BlockSpec((1, 64), lambda i: (0, 0)),
            pl.BlockSpec((1, 128), lambda i: (0, 0)),
        ],
        out_specs=pl.BlockSpec((1, 128, 64), lambda i: (i, 0, 0)),
        scratch_shapes=[
            pltpu.VMEM((4096 + 144, 288), F32),
            pltpu.VMEM((1024 + 80, 576), F32),
            pltpu.VMEM((256 + 48, 1152), F32),
        ],
        compiler_params=pltpu.CompilerParams(
            dimension_semantics=("parallel",),
            vmem_limit_bytes=100 * 1024 * 1024,
        ),
        name="cae_encoder",
    )(x4, w1e, w2e, w3e,
      c1b.reshape(1, 32), c2b.reshape(1, 64), c3b.reshape(1, 128))

    h = h3.reshape(B, 8192)

    lat_pad = pl.pallas_call(
        _fc_kernel,
        out_shape=jax.ShapeDtypeStruct((B, 128), F32),
        grid=(8,),
        in_specs=[
            pl.BlockSpec((B, 8192), lambda j: (0, 0)),
            pl.BlockSpec((256, 8192), lambda j: (j, 0)),
            pl.BlockSpec((1, 256), lambda j: (0, j)),
            pl.BlockSpec((1024, 2048), lambda j: (0, 0)),
            pl.BlockSpec((1, 1024), lambda j: (0, 0)),
            pl.BlockSpec((512, 1024), lambda j: (0, 0)),
            pl.BlockSpec((1, 512), lambda j: (0, 0)),
            pl.BlockSpec((256, 512), lambda j: (0, 0)),
            pl.BlockSpec((1, 256), lambda j: (0, 0)),
            pl.BlockSpec((128, 256), lambda j: (0, 0)),
            pl.BlockSpec((1, 128), lambda j: (0, 0)),
        ],
        out_specs=pl.BlockSpec((B, 128), lambda j: (0, 0)),
        scratch_shapes=[pltpu.VMEM((B, 2048), F32)],
        compiler_params=pltpu.CompilerParams(
            dimension_semantics=("arbitrary",),
            vmem_limit_bytes=100 * 1024 * 1024,
        ),
        name="cae_fc",
    )(h, l1w, l1b.reshape(1, 2048), l2w, l2b.reshape(1, 1024),
      l3w, l3b.reshape(1, 512), l4w, l4b.reshape(1, 256),
      jnp.pad(l5w, ((0, 122), (0, 0))), jnp.pad(l5b, (0, 122)).reshape(1, 128))

    lat = lat_pad[:, :6]
    z0, z1, inc = lat[:, 0], lat[:, 1], lat[:, 2]
    pos = jnp.arctan2(z0, z1)
    derived = jnp.stack([jnp.cos(pos), jnp.sin(pos), jnp.cos(inc),
                         jnp.sin(inc), jnp.abs(lat[:, 3]), lat[:, 4],
                         lat[:, 5], jnp.zeros_like(z0)], axis=1)
    derived = derived.reshape(B, 1, 8)

    cube, v = pl.pallas_call(
        _cube_kernel,
        out_shape=(jax.ShapeDtypeStruct((B, V_SIZE + 1, 32, 128), F32),
                   jax.ShapeDtypeStruct((B, 32, 128), F32)),
        grid=(B,),
        in_specs=[
            pl.BlockSpec((1, 1, 8), lambda i: (i, 0, 0)),
            pl.BlockSpec((32, 128), lambda i: (0, 0)),
            pl.BlockSpec((32, 128), lambda i: (0, 0)),
        ],
        out_specs=(pl.BlockSpec((1, V_SIZE + 1, 32, 128), lambda i: (i, 0, 0, 0)),
                   pl.BlockSpec((1, 32, 128), lambda i: (i, 0, 0))),
        compiler_params=pltpu.CompilerParams(
            dimension_semantics=("parallel",),
            vmem_limit_bytes=100 * 1024 * 1024,
        ),
        name="cae_cube",
    )(derived, xx.reshape(32, 128), yy.reshape(32, 128))

    return cube.reshape(B, V_SIZE + 1, 64, 64), v.reshape(B, 64, 64)
